# baseline (speedup 1.0000x reference)
import jax
import jax.numpy as jnp
from jax import lax
from jax.experimental import pallas as pl
from jax.experimental.pallas import tpu as pltpu
from jax.experimental.pallas import tpu_sc as plsc

NC = 2
NS = 16
NW = NC * NS
LANES = 16
EMBED = 64
BLK = 128
NBUF = 4
SCALE = 8.0


def _make_sc_kernel(b, t):
    mesh = plsc.VectorSubcoreMesh(core_axis_name="c", subcore_axis_name="s")
    n_blk = b // BLK
    assert n_blk == NW
    n_units = t

    def body(idx_hbm, table_hbm, out_hbm, idx_v, *bufs):
        in_v = bufs[:NBUF]
        out_v = bufs[NBUF:2 * NBUF]
        gsems = bufs[2 * NBUF:3 * NBUF]
        ssems = bufs[3 * NBUF:4 * NBUF]
        wid = lax.axis_index("s") * NC + lax.axis_index("c")
        pltpu.sync_copy(idx_hbm.at[:, pl.ds(wid * BLK, BLK)], idx_v)

        def gather(u, slot):
            pltpu.async_copy(table_hbm.at[idx_v.at[u]], in_v[slot], gsems[slot])

        def gather_wait(u, slot):
            pltpu.make_async_copy(
                table_hbm.at[idx_v.at[u]], in_v[slot], gsems[slot]).wait()

        def scatter(u, slot):
            for dk in range(EMBED // 8):
                pltpu.async_copy(out_v[slot].at[dk], out_hbm.at[u, dk, wid],
                                 ssems[slot])

        def scatter_wait(u, slot):
            for dk in range(EMBED // 8):
                pltpu.make_async_copy(
                    out_v[slot].at[dk], out_hbm.at[u, dk, wid],
                    ssems[slot]).wait()

        for slot in range(NBUF):
            gather(slot, slot)

        n_groups = n_units // NBUF

        def group_body(g, carry):
            for slot in range(NBUF):
                u = g * NBUF + slot
                gather_wait(u, slot)

                @pl.when(g >= 1)
                def _():
                    scatter_wait(u - NBUF, slot)

                @plsc.parallel_loop(0, EMBED, 1, unroll=2)
                def _(d):
                    dk = d // 8
                    ds_ = d - dk * 8
                    base = lax.iota(jnp.int32, 16)
                    col = jnp.broadcast_to(d, (16,)).astype(jnp.int32)
                    for tg in range(BLK // LANES):
                        rows = base + (tg * LANES)
                        v = plsc.load_gather(in_v[slot], [rows, col])
                        out_v[slot][dk, ds_, pl.ds(tg * LANES, LANES)] = v * SCALE

                @pl.when(g < n_groups - 1)
                def _():
                    gather(u + NBUF, slot)

                scatter(u, slot)
            return carry

        lax.fori_loop(0, n_groups, group_body, 0)

        for slot in range(NBUF):
            scatter_wait(n_units - NBUF + slot, slot)

    return pl.kernel(
        body,
        out_type=jax.ShapeDtypeStruct((t, EMBED // 8, NW, 8, BLK), jnp.float32),
        mesh=mesh,
        scratch_types=(
            [pltpu.VMEM((n_units, BLK), jnp.int32)]
            + [pltpu.VMEM((BLK, EMBED), jnp.float32)] * NBUF
            + [pltpu.VMEM((EMBED // 8, 8, BLK), jnp.float32)] * NBUF
            + [pltpu.SemaphoreType.DMA] * (2 * NBUF)
        ),
        compiler_params=pltpu.CompilerParams(use_tc_tiling_on_sc=False,
                                             needs_layout_passes=False),
    )


def kernel(input, table):
    b, t = input.shape
    idx_t = input.T.astype(jnp.int32)
    out5 = _make_sc_kernel(b, t)(idx_t, table)
    return out5.transpose(2, 4, 0, 1, 3).reshape(b, t, EMBED)


# speedup vs baseline: 1.0029x; 1.0029x over previous
import jax
import jax.numpy as jnp
from jax import lax
from jax.experimental import pallas as pl
from jax.experimental.pallas import tpu as pltpu
from jax.experimental.pallas import tpu_sc as plsc

NC = 2
NS = 16
NW = NC * NS
LANES = 16
EMBED = 64
BLK = 128
NBUF = 4
SCALE = 8.0


def _make_sc_kernel(b, t):
    mesh = plsc.VectorSubcoreMesh(core_axis_name="c", subcore_axis_name="s")
    n_blk = b // BLK
    assert n_blk == NW
    n_tr = t // 8
    n_units = t

    def body(idx_hbm, table_hbm, out_hbm, idx_v, *bufs):
        in_v = bufs[:NBUF]
        out_v = bufs[NBUF:2 * NBUF]
        gsems = bufs[2 * NBUF:3 * NBUF]
        ssems = bufs[3 * NBUF:4 * NBUF]
        wid = lax.axis_index("s") * NC + lax.axis_index("c")
        pltpu.sync_copy(idx_hbm.at[:, wid], idx_v)

        def gather(u, slot):
            i = u // 8
            s = u - i * 8
            pltpu.async_copy(table_hbm.at[idx_v.at[i, s]], in_v[slot],
                             gsems[slot])

        def gather_wait(slot):
            pltpu.make_async_copy(
                table_hbm.at[idx_v.at[0, 0]], in_v[slot], gsems[slot]).wait()

        def scatter(u, slot):
            for dk in range(EMBED // 8):
                pltpu.async_copy(out_v[slot].at[pl.ds(dk * 8, 8)],
                                 out_hbm.at[u, dk, wid], ssems[slot])

        def scatter_wait(u, slot):
            for dk in range(EMBED // 8):
                pltpu.make_async_copy(
                    out_v[slot].at[pl.ds(dk * 8, 8)],
                    out_hbm.at[u, dk, wid], ssems[slot]).wait()

        base = lax.iota(jnp.int32, 16)
        row_ids = [base + (tg * LANES) for tg in range(BLK // LANES)]

        for slot in range(NBUF):
            gather(slot, slot)

        n_groups = n_units // NBUF

        def group_body(g, carry):
            for slot in range(NBUF):
                u = g * NBUF + slot
                gather_wait(slot)

                @pl.when(g >= 1)
                def _():
                    scatter_wait(u - NBUF, slot)

                @plsc.parallel_loop(0, EMBED, 1, unroll=4)
                def _(d):
                    col = jnp.broadcast_to(d, (16,)).astype(jnp.int32)
                    for tg in range(BLK // LANES):
                        v = plsc.load_gather(in_v[slot], [row_ids[tg], col])
                        out_v[slot][d, pl.ds(tg * LANES, LANES)] = v * SCALE

                @pl.when(g < n_groups - 1)
                def _():
                    gather(u + NBUF, slot)

                scatter(u, slot)
            return carry

        lax.fori_loop(0, n_groups, group_body, 0)

        for slot in range(NBUF):
            scatter_wait(n_units - NBUF + slot, slot)

    return pl.kernel(
        body,
        out_type=jax.ShapeDtypeStruct((t, EMBED // 8, NW, 8, BLK), jnp.float32),
        mesh=mesh,
        scratch_types=(
            [pltpu.VMEM((n_tr, 8, BLK), jnp.int32)]
            + [pltpu.VMEM((BLK, EMBED), jnp.float32)] * NBUF
            + [pltpu.VMEM((EMBED, BLK), jnp.float32)] * NBUF
            + [pltpu.SemaphoreType.DMA] * (2 * NBUF)
        ),
        compiler_params=pltpu.CompilerParams(use_tc_tiling_on_sc=False,
                                             needs_layout_passes=False),
    )


def kernel(input, table):
    b, t = input.shape
    idx_view = (input.astype(jnp.int32).T
                .reshape(t // 8, 8, b // BLK, BLK)
                .transpose(0, 2, 1, 3))
    out5 = _make_sc_kernel(b, t)(idx_view, table)
    return out5.transpose(2, 4, 0, 1, 3).reshape(b, t, EMBED)


# speedup vs baseline: 1.6875x; 1.6826x over previous
import jax
import jax.numpy as jnp
from jax import lax
from jax.experimental import pallas as pl
from jax.experimental.pallas import tpu as pltpu
from jax.experimental.pallas import tpu_sc as plsc

NC = 2
NS = 16
NW = NC * NS
LANES = 16
EMBED = 64
BLK = 128
NBUF = 4
SCALE = 8.0


def _make_sc_kernel(b, t):
    mesh = plsc.VectorSubcoreMesh(core_axis_name="c", subcore_axis_name="s")
    n_blk = b // BLK
    assert n_blk == NW
    n_tr = t // 8
    n_units = t

    def body(idx_hbm, table_hbm, out_hbm, idx_v, *bufs):
        in_v = bufs[:NBUF]
        out_v = bufs[NBUF:2 * NBUF]
        gsems = bufs[2 * NBUF:3 * NBUF]
        ssems = bufs[3 * NBUF:4 * NBUF]
        wid = lax.axis_index("s") * NC + lax.axis_index("c")
        pltpu.sync_copy(idx_hbm.at[:, wid], idx_v)

        def gather(u, slot):
            i = u // 8
            s = u - i * 8
            pltpu.async_copy(table_hbm.at[idx_v.at[i, s]], in_v[slot],
                             gsems[slot])

        def gather_wait(slot):
            pltpu.make_async_copy(
                table_hbm.at[idx_v.at[0, 0]], in_v[slot], gsems[slot]).wait()

        def scatter(u, slot):
            for dk in range(EMBED // 8):
                pltpu.async_copy(out_v[slot].at[pl.ds(dk * 8, 8), pl.ds(0, BLK)],
                                 out_hbm.at[u, dk, wid], ssems[slot])

        def scatter_wait(u, slot):
            for dk in range(EMBED // 8):
                pltpu.make_async_copy(
                    out_v[slot].at[pl.ds(dk * 8, 8), pl.ds(0, BLK)],
                    out_hbm.at[u, dk, wid], ssems[slot]).wait()

        base = lax.iota(jnp.int32, 16)
        d_ids = [base + (k * LANES) for k in range(EMBED // LANES)]

        for slot in range(NBUF):
            gather(slot, slot)

        n_groups = n_units // NBUF

        def group_body(g, carry):
            for slot in range(NBUF):
                u = g * NBUF + slot
                gather_wait(slot)

                @pl.when(g >= 1)
                def _():
                    scatter_wait(u - NBUF, slot)

                @plsc.parallel_loop(0, BLK, 1, unroll=2)
                def _(l):
                    tok = jnp.broadcast_to(l, (16,)).astype(jnp.int32)
                    for k in range(EMBED // LANES):
                        v = in_v[slot][l, pl.ds(k * LANES, LANES)]
                        plsc.store_scatter(out_v[slot], [d_ids[k], tok],
                                           v * SCALE)

                @pl.when(g < n_groups - 1)
                def _():
                    gather(u + NBUF, slot)

                scatter(u, slot)
            return carry

        lax.fori_loop(0, n_groups, group_body, 0)

        for slot in range(NBUF):
            scatter_wait(n_units - NBUF + slot, slot)

    return pl.kernel(
        body,
        out_type=jax.ShapeDtypeStruct((t, EMBED // 8, NW, 8, BLK), jnp.float32),
        mesh=mesh,
        scratch_types=(
            [pltpu.VMEM((n_tr, 8, BLK), jnp.int32)]
            + [pltpu.VMEM((BLK, EMBED), jnp.float32)] * NBUF
            + [pltpu.VMEM((EMBED, BLK + 5), jnp.float32)] * NBUF
            + [pltpu.SemaphoreType.DMA] * (2 * NBUF)
        ),
        compiler_params=pltpu.CompilerParams(use_tc_tiling_on_sc=False,
                                             needs_layout_passes=False),
    )


def kernel(input, table):
    b, t = input.shape
    idx_view = (input.astype(jnp.int32).T
                .reshape(t // 8, 8, b // BLK, BLK)
                .transpose(0, 2, 1, 3))
    out5 = _make_sc_kernel(b, t)(idx_view, table)
    return out5.transpose(2, 4, 0, 1, 3).reshape(b, t, EMBED)
